# baseline (speedup 1.0000x reference)
import functools

import jax
import jax.numpy as jnp
from jax import lax
from jax.experimental import pallas as pl
from jax.experimental.pallas import tpu as pltpu
from jax.experimental.pallas import tpu_sc as plsc

B, T, D, E, K = 1, 2048, 1024, 8, 2
N = T * K
BLK = 256
NT_MAX = N // BLK + E
NPAD = NT_MAX * BLK
NF = 4
F = 4 * D
FBLK = F // NF

NW = 32
CHUNK = NPAD // NW
GB = 48
NGB = CHUNK // GB
TCH = T // NW
SCH = 16


def _prep(expert_indices):
    e_flat = expert_indices.reshape(N).astype(jnp.int32)
    onehot = e_flat[:, None] == jnp.arange(E, dtype=jnp.int32)[None, :]
    counts = jnp.sum(onehot, axis=0, dtype=jnp.int32)
    rank = jnp.cumsum(onehot.astype(jnp.int32), axis=0)
    rank_i = jnp.take_along_axis(rank, e_flat[:, None], axis=1)[:, 0] - 1
    tiles_per_e = (counts + BLK - 1) // BLK
    tile_start_e = jnp.concatenate(
        [jnp.zeros(1, jnp.int32), jnp.cumsum(tiles_per_e, dtype=jnp.int32)])[:E]
    P = tile_start_e[e_flat] * BLK + rank_i
    tok = jnp.arange(N, dtype=jnp.int32) // K
    tok_padded = jnp.zeros(NPAD, jnp.int32).at[P].set(tok)
    pairid_padded = jnp.zeros(NPAD, jnp.int32).at[P].set(
        jnp.arange(N, dtype=jnp.int32))
    total_tiles = jnp.sum(tiles_per_e)
    tile_ids = jnp.arange(NT_MAX, dtype=jnp.int32)
    tile_expert = jnp.sum(tile_ids[:, None] >= tile_start_e[None, :],
                          axis=1, dtype=jnp.int32) - 1
    tile_expert = jnp.where(tile_ids < total_tiles, tile_expert, -1)
    p0 = P.reshape(T, K)[:, 0]
    p1 = P.reshape(T, K)[:, 1]
    return tok_padded, pairid_padded, tile_expert, p0, p1


def _ffn_body(te_ref, xs_ref, w1_ref, b1_ref, w2_ref, b2_ref, ws_ref,
              out_ref, acc_ref):
    fi = pl.program_id(1)
    j = pl.program_id(0)
    active = te_ref[j] >= 0

    @pl.when(active)
    def _():
        h = jnp.dot(xs_ref[...], w1_ref[0], preferred_element_type=jnp.float32)
        h = h + b1_ref[0, 0, 0][None, :]
        h = h * 0.5 * (1.0 + lax.erf(h * 0.7071067811865476))
        y = jnp.dot(h, w2_ref[0], preferred_element_type=jnp.float32)

        @pl.when(fi == 0)
        def _():
            acc_ref[...] = y

        @pl.when(fi > 0)
        def _():
            acc_ref[...] += y

        @pl.when(fi == NF - 1)
        def _():
            out_ref[...] = ((acc_ref[...] + b2_ref[0, 0][None, :])
                            * ws_ref[0, 0][:, None])


def _grouped_ffn(tile_expert, xs, W1, b1, W2, b2, wsorted):
    b1r = b1.reshape(E, NF, 1, FBLK)
    b2r = b2.reshape(E, 1, D)
    wsr = wsorted.reshape(NT_MAX, 1, BLK)
    grid_spec = pltpu.PrefetchScalarGridSpec(
        num_scalar_prefetch=1,
        grid=(NT_MAX, NF),
        in_specs=[
            pl.BlockSpec((BLK, D), lambda j, fi, te: (j, 0)),
            pl.BlockSpec((1, D, FBLK), lambda j, fi, te: (jnp.maximum(te[j], 0), 0, fi)),
            pl.BlockSpec((1, 1, 1, FBLK), lambda j, fi, te: (jnp.maximum(te[j], 0), fi, 0, 0)),
            pl.BlockSpec((1, FBLK, D), lambda j, fi, te: (jnp.maximum(te[j], 0), fi, 0)),
            pl.BlockSpec((1, 1, D), lambda j, fi, te: (jnp.maximum(te[j], 0), 0, 0)),
            pl.BlockSpec((1, 1, BLK), lambda j, fi, te: (j, 0, 0)),
        ],
        out_specs=pl.BlockSpec((BLK, D), lambda j, fi, te: (j, 0)),
        scratch_shapes=[pltpu.VMEM((BLK, D), jnp.float32)],
    )
    return pl.pallas_call(
        _ffn_body,
        grid_spec=grid_spec,
        out_shape=jax.ShapeDtypeStruct((NPAD, D), jnp.float32),
    )(tile_expert, xs, W1, b1r, W2, b2r, wsr)


_SC_MESH = plsc.VectorSubcoreMesh(core_axis_name="c", subcore_axis_name="s")


@functools.partial(
    pl.kernel,
    mesh=_SC_MESH,
    out_type=[
        jax.ShapeDtypeStruct((NPAD, D), jnp.float32),
        jax.ShapeDtypeStruct((NPAD,), jnp.float32),
    ],
    scratch_types=[
        pltpu.VMEM((NGB, GB), jnp.int32),
        pltpu.VMEM((CHUNK,), jnp.int32),
        pltpu.VMEM((CHUNK,), jnp.int32),
        pltpu.VMEM((CHUNK,), jnp.float32),
        pltpu.VMEM((CHUNK,), jnp.float32),
        pltpu.VMEM((CHUNK,), jnp.float32),
        pltpu.VMEM((GB, D), jnp.float32),
        pltpu.VMEM((GB, D), jnp.float32),
        pltpu.SemaphoreType.DMA,
        pltpu.SemaphoreType.DMA,
    ],
)
def _sc_dispatch(x_hbm, tok_hbm, pid_hbm, wf_hbm, xs_hbm, ws_hbm,
                 idx_v, pid_v, sib_v, wa_v, wb_v, ws_v,
                 rows_a, rows_b, sem_a, sem_b):
    wid = lax.axis_index("s") * 2 + lax.axis_index("c")
    base = wid * CHUNK
    pltpu.sync_copy(tok_hbm.at[wid], idx_v)
    pltpu.sync_copy(pid_hbm.at[pl.ds(base, CHUNK)], pid_v)
    for i in range(CHUNK // 16):
        cs = pl.ds(i * 16, 16)
        sib_v[cs] = pid_v[cs] ^ 1
    ga = pltpu.async_copy(wf_hbm.at[pid_v], wa_v, sem_a)
    gb = pltpu.async_copy(wf_hbm.at[sib_v], wb_v, sem_b)
    ga.wait()
    gb.wait()
    for i in range(CHUNK // 16):
        cs = pl.ds(i * 16, 16)
        wa = wa_v[cs]
        wb = wb_v[cs]
        m = jnp.maximum(wa, wb)
        ea = jnp.exp(wa - m)
        eb = jnp.exp(wb - m)
        ws_v[cs] = ea / (ea + eb)
    pltpu.sync_copy(ws_v, ws_hbm.at[pl.ds(base, CHUNK)])
    bufs = (rows_a, rows_b)
    sems = (sem_a, sem_b)
    cps = [pltpu.async_copy(x_hbm.at[idx_v.at[0]], bufs[0], sems[0])]
    for g in range(NGB):
        cps[g].wait()
        if g + 1 < NGB:
            cps.append(pltpu.async_copy(
                x_hbm.at[idx_v.at[g + 1]], bufs[(g + 1) % 2], sems[(g + 1) % 2]))
        pltpu.sync_copy(bufs[g % 2], xs_hbm.at[pl.ds(base + g * GB, GB)])


@functools.partial(
    pl.kernel,
    mesh=_SC_MESH,
    out_type=jax.ShapeDtypeStruct((T, D), jnp.float32),
    scratch_types=[
        pltpu.VMEM((TCH // SCH, SCH), jnp.int32),
        pltpu.VMEM((TCH // SCH, SCH), jnp.int32),
        pltpu.VMEM((SCH, D), jnp.float32),
        pltpu.VMEM((SCH, D), jnp.float32),
        pltpu.VMEM((SCH, D), jnp.float32),
        pltpu.VMEM((SCH, D), jnp.float32),
        pltpu.SemaphoreType.DMA,
        pltpu.SemaphoreType.DMA,
    ],
)
def _sc_combine(x_hbm, ysw_hbm, p0_hbm, p1_hbm, out_hbm,
                p0_v, p1_v, rows0, rows1, xb, ob, sem0, sem1):
    wid = lax.axis_index("s") * 2 + lax.axis_index("c")
    base = wid * TCH
    pltpu.sync_copy(p0_hbm.at[wid], p0_v)
    pltpu.sync_copy(p1_hbm.at[wid], p1_v)
    for c in range(TCH // SCH):
        tb = base + c * SCH
        g0 = pltpu.async_copy(ysw_hbm.at[p0_v.at[c]], rows0, sem0)
        g1 = pltpu.async_copy(ysw_hbm.at[p1_v.at[c]], rows1, sem1)
        pltpu.sync_copy(x_hbm.at[pl.ds(tb, SCH)], xb)
        g0.wait()
        g1.wait()

        def body(t, _):
            for col in range(D // 16):
                cs = pl.ds(col * 16, 16)
                ob[t, cs] = xb[t, cs] + rows0[t, cs] + rows1[t, cs]
            return 0

        lax.fori_loop(0, SCH, body, 0)
        pltpu.sync_copy(ob, out_hbm.at[pl.ds(tb, SCH)])


def kernel(x, expert_indices, expert_weights, W1, b1, W2, b2):
    x2d = x.reshape(T, D)
    tok_padded, pairid_padded, tile_expert, p0, p1 = _prep(expert_indices)

    xs, wsorted = _sc_dispatch(
        x2d, tok_padded.reshape(NW, NGB, GB), pairid_padded,
        expert_weights.reshape(N))

    ysw = _grouped_ffn(tile_expert, xs, W1, b1, W2, b2, wsorted)

    out = _sc_combine(x2d, ysw,
                      p0.reshape(NW, TCH // SCH, SCH),
                      p1.reshape(NW, TCH // SCH, SCH))
    return out.reshape(B, T, D)


# speedup vs baseline: 1.0149x; 1.0149x over previous
import functools

import jax
import jax.numpy as jnp
from jax import lax
from jax.experimental import pallas as pl
from jax.experimental.pallas import tpu as pltpu
from jax.experimental.pallas import tpu_sc as plsc

B, T, D, E, K = 1, 2048, 1024, 8, 2
N = T * K
BLK = 256
NT_MAX = N // BLK + E
NPAD = NT_MAX * BLK
NF = 4
F = 4 * D
FBLK = F // NF

NW = 32
CHUNK = NPAD // NW
GB = 48
NGB = CHUNK // GB
TCH = T // NW
SCH = 16


def _prep(expert_indices):
    e_flat = expert_indices.reshape(N).astype(jnp.int32)
    onehot = e_flat[:, None] == jnp.arange(E, dtype=jnp.int32)[None, :]
    counts = jnp.sum(onehot, axis=0, dtype=jnp.int32)
    rank = jnp.cumsum(onehot.astype(jnp.int32), axis=0)
    rank_i = jnp.take_along_axis(rank, e_flat[:, None], axis=1)[:, 0] - 1
    tiles_per_e = (counts + BLK - 1) // BLK
    tile_start_e = jnp.concatenate(
        [jnp.zeros(1, jnp.int32), jnp.cumsum(tiles_per_e, dtype=jnp.int32)])[:E]
    P = tile_start_e[e_flat] * BLK + rank_i
    tok = jnp.arange(N, dtype=jnp.int32) // K
    tok_padded = jnp.zeros(NPAD, jnp.int32).at[P].set(tok)
    pairid_padded = jnp.zeros(NPAD, jnp.int32).at[P].set(
        jnp.arange(N, dtype=jnp.int32))
    total_tiles = jnp.sum(tiles_per_e)
    tile_ids = jnp.arange(NT_MAX, dtype=jnp.int32)
    tile_expert = jnp.sum(tile_ids[:, None] >= tile_start_e[None, :],
                          axis=1, dtype=jnp.int32) - 1
    tile_expert = jnp.where(tile_ids < total_tiles, tile_expert, -1)
    p0 = P.reshape(T, K)[:, 0]
    p1 = P.reshape(T, K)[:, 1]
    return tok_padded, pairid_padded, tile_expert, p0, p1


def _ffn_body(te_ref, xs_ref, w1_ref, b1_ref, w2_ref, b2_ref, ws_ref,
              out_ref, acc_ref):
    fi = pl.program_id(1)
    j = pl.program_id(0)
    active = te_ref[j] >= 0

    @pl.when(active)
    def _():
        h = jnp.dot(xs_ref[...], w1_ref[0], preferred_element_type=jnp.float32)
        h = h + b1_ref[0, 0, 0][None, :]
        h = h * 0.5 * (1.0 + lax.erf(h * 0.7071067811865476))
        y = jnp.dot(h, w2_ref[0], preferred_element_type=jnp.float32)

        @pl.when(fi == 0)
        def _():
            acc_ref[...] = y

        @pl.when(fi > 0)
        def _():
            acc_ref[...] += y

        @pl.when(fi == NF - 1)
        def _():
            out_ref[...] = ((acc_ref[...] + b2_ref[0, 0][None, :])
                            * ws_ref[0, 0][:, None])


def _grouped_ffn(tile_expert, xs, W1, b1, W2, b2, wsorted):
    b1r = b1.reshape(E, NF, 1, FBLK)
    b2r = b2.reshape(E, 1, D)
    wsr = wsorted.reshape(NT_MAX, 1, BLK)
    grid_spec = pltpu.PrefetchScalarGridSpec(
        num_scalar_prefetch=1,
        grid=(NT_MAX, NF),
        in_specs=[
            pl.BlockSpec((BLK, D), lambda j, fi, te: (j, 0)),
            pl.BlockSpec((1, D, FBLK), lambda j, fi, te: (jnp.maximum(te[j], 0), 0, fi)),
            pl.BlockSpec((1, 1, 1, FBLK), lambda j, fi, te: (jnp.maximum(te[j], 0), fi, 0, 0)),
            pl.BlockSpec((1, FBLK, D), lambda j, fi, te: (jnp.maximum(te[j], 0), fi, 0)),
            pl.BlockSpec((1, 1, D), lambda j, fi, te: (jnp.maximum(te[j], 0), 0, 0)),
            pl.BlockSpec((1, 1, BLK), lambda j, fi, te: (j, 0, 0)),
        ],
        out_specs=pl.BlockSpec((BLK, D), lambda j, fi, te: (j, 0)),
        scratch_shapes=[pltpu.VMEM((BLK, D), jnp.float32)],
    )
    return pl.pallas_call(
        _ffn_body,
        grid_spec=grid_spec,
        out_shape=jax.ShapeDtypeStruct((NPAD, D), jnp.float32),
    )(tile_expert, xs, W1, b1r, W2, b2r, wsr)


_SC_MESH = plsc.VectorSubcoreMesh(core_axis_name="c", subcore_axis_name="s")


@functools.partial(
    pl.kernel,
    mesh=_SC_MESH,
    out_type=[
        jax.ShapeDtypeStruct((NPAD, D), jnp.float32),
        jax.ShapeDtypeStruct((NPAD,), jnp.float32),
    ],
    scratch_types=[
        pltpu.VMEM((NGB, GB), jnp.int32),
        pltpu.VMEM((CHUNK,), jnp.int32),
        pltpu.VMEM((CHUNK,), jnp.float32),
        pltpu.VMEM((CHUNK,), jnp.float32),
        pltpu.VMEM((GB, D), jnp.float32),
        pltpu.VMEM((GB, D), jnp.float32),
        pltpu.SemaphoreType.DMA,
        pltpu.SemaphoreType.DMA,
        pltpu.SemaphoreType.DMA,
    ],
)
def _sc_dispatch(x_hbm, tok_hbm, pid_hbm, df_hbm, xs_hbm, ws_hbm,
                 idx_v, pid_v, da_v, ws_v, rows_a, rows_b, sem_a, sem_b, sem_w):
    wid = lax.axis_index("s") * 2 + lax.axis_index("c")
    base = wid * CHUNK
    pltpu.sync_copy(tok_hbm.at[wid], idx_v)
    bufs = (rows_a, rows_b)
    sems = (sem_a, sem_b)
    cps = [pltpu.async_copy(x_hbm.at[idx_v.at[0]], bufs[0], sems[0]),
           pltpu.async_copy(x_hbm.at[idx_v.at[1]], bufs[1], sems[1])]
    pltpu.sync_copy(pid_hbm.at[pl.ds(base, CHUNK)], pid_v)
    gw = pltpu.async_copy(df_hbm.at[pid_v], da_v, sem_w)
    gw.wait()
    for i in range(CHUNK // 16):
        cs = pl.ds(i * 16, 16)
        ws_v[cs] = 1.0 / (1.0 + jnp.exp(-da_v[cs]))
    pltpu.sync_copy(ws_v, ws_hbm.at[pl.ds(base, CHUNK)])
    for g in range(NGB):
        cps[g].wait()
        pltpu.sync_copy(bufs[g % 2], xs_hbm.at[pl.ds(base + g * GB, GB)])
        if g + 2 < NGB:
            cps.append(pltpu.async_copy(
                x_hbm.at[idx_v.at[g + 2]], bufs[g % 2], sems[g % 2]))


@functools.partial(
    pl.kernel,
    mesh=_SC_MESH,
    out_type=jax.ShapeDtypeStruct((T, D), jnp.float32),
    scratch_types=[
        pltpu.VMEM((TCH // SCH, SCH), jnp.int32),
        pltpu.VMEM((TCH // SCH, SCH), jnp.int32),
        pltpu.VMEM((2, SCH, D), jnp.float32),
        pltpu.VMEM((2, SCH, D), jnp.float32),
        pltpu.VMEM((2, SCH, D), jnp.float32),
        pltpu.VMEM((SCH, D), jnp.float32),
        pltpu.SemaphoreType.DMA,
        pltpu.SemaphoreType.DMA,
    ],
)
def _sc_combine(x_hbm, ysw_hbm, p0_hbm, p1_hbm, out_hbm,
                p0_v, p1_v, rows0, rows1, xb, ob, sem0, sem1):
    wid = lax.axis_index("s") * 2 + lax.axis_index("c")
    base = wid * TCH
    pltpu.sync_copy(p0_hbm.at[wid], p0_v)
    pltpu.sync_copy(p1_hbm.at[wid], p1_v)
    nc = TCH // SCH
    sems = (sem0, sem1)

    def fire(c):
        buf = c % 2
        return [pltpu.async_copy(ysw_hbm.at[p0_v.at[c]], rows0.at[buf], sems[buf]),
                pltpu.async_copy(ysw_hbm.at[p1_v.at[c]], rows1.at[buf], sems[buf]),
                pltpu.async_copy(x_hbm.at[pl.ds(base + c * SCH, SCH)],
                                 xb.at[buf], sems[buf])]

    pend = {0: fire(0), 1: fire(1)}
    for c in range(nc):
        buf = c % 2
        for cp in pend.pop(c):
            cp.wait()

        def body(t, _):
            for col in range(D // 16):
                cs = pl.ds(col * 16, 16)
                ob[t, cs] = xb[buf, t, cs] + rows0[buf, t, cs] + rows1[buf, t, cs]
            return 0

        lax.fori_loop(0, SCH, body, 0)
        pltpu.sync_copy(ob, out_hbm.at[pl.ds(base + c * SCH, SCH)])
        if c + 2 < nc:
            pend[c + 2] = fire(c + 2)


def kernel(x, expert_indices, expert_weights, W1, b1, W2, b2):
    x2d = x.reshape(T, D)
    tok_padded, pairid_padded, tile_expert, p0, p1 = _prep(expert_indices)

    wdiff = (expert_weights - expert_weights[..., ::-1]).reshape(N)
    xs, wsorted = _sc_dispatch(
        x2d, tok_padded.reshape(NW, NGB, GB), pairid_padded, wdiff)

    ysw = _grouped_ffn(tile_expert, xs, W1, b1, W2, b2, wsorted)

    out = _sc_combine(x2d, ysw,
                      p0.reshape(NW, TCH // SCH, SCH),
                      p1.reshape(NW, TCH // SCH, SCH))
    return out.reshape(B, T, D)


# speedup vs baseline: 1.0492x; 1.0338x over previous
import functools

import jax
import jax.numpy as jnp
from jax import lax
from jax.experimental import pallas as pl
from jax.experimental.pallas import tpu as pltpu
from jax.experimental.pallas import tpu_sc as plsc

B, T, D, E, K = 1, 2048, 1024, 8, 2
N = T * K
BLK = 256
NT_MAX = N // BLK + E
NPAD = NT_MAX * BLK
NF = 4
F = 4 * D
FBLK = F // NF

NW = 32
CHUNK = NPAD // NW
GB = 48
NGB = CHUNK // GB
TCH = T // NW
SCH = 16
L = 16


def _prep(expert_indices):
    e_flat = expert_indices.reshape(N).astype(jnp.int32)
    onehot = e_flat[:, None] == jnp.arange(E, dtype=jnp.int32)[None, :]
    counts = jnp.sum(onehot, axis=0, dtype=jnp.int32)
    rank = jnp.cumsum(onehot.astype(jnp.int32), axis=0)
    rank_i = jnp.take_along_axis(rank, e_flat[:, None], axis=1)[:, 0] - 1
    tiles_per_e = (counts + BLK - 1) // BLK
    tile_start_e = jnp.concatenate(
        [jnp.zeros(1, jnp.int32), jnp.cumsum(tiles_per_e, dtype=jnp.int32)])[:E]
    P = tile_start_e[e_flat] * BLK + rank_i
    tok = jnp.arange(N, dtype=jnp.int32) // K
    tok_padded = jnp.zeros(NPAD, jnp.int32).at[P].set(tok)
    total_tiles = jnp.sum(tiles_per_e)
    tile_ids = jnp.arange(NT_MAX, dtype=jnp.int32)
    tile_expert = jnp.sum(tile_ids[:, None] >= tile_start_e[None, :],
                          axis=1, dtype=jnp.int32) - 1
    tile_expert = jnp.where(tile_ids < total_tiles, tile_expert, -1)
    p0 = P.reshape(T, K)[:, 0]
    p1 = P.reshape(T, K)[:, 1]
    return tok_padded, tile_expert, p0, p1


def _ffn_body(te_ref, xs_ref, w1_ref, b1_ref, w2_ref, b2_ref,
              out_ref, acc_ref):
    fi = pl.program_id(1)
    j = pl.program_id(0)
    active = te_ref[j] >= 0

    @pl.when(active)
    def _():
        h = jnp.dot(xs_ref[...], w1_ref[0], preferred_element_type=jnp.float32)
        h = h + b1_ref[0, 0, 0][None, :]
        h = h * 0.5 * (1.0 + lax.erf(h * 0.7071067811865476))
        y = jnp.dot(h, w2_ref[0], preferred_element_type=jnp.float32)

        @pl.when(fi == 0)
        def _():
            acc_ref[...] = y

        @pl.when(fi > 0)
        def _():
            acc_ref[...] += y

        @pl.when(fi == NF - 1)
        def _():
            out_ref[...] = acc_ref[...] + b2_ref[0, 0][None, :]


def _grouped_ffn(tile_expert, xs, W1, b1, W2, b2):
    b1r = b1.reshape(E, NF, 1, FBLK)
    b2r = b2.reshape(E, 1, D)
    grid_spec = pltpu.PrefetchScalarGridSpec(
        num_scalar_prefetch=1,
        grid=(NT_MAX, NF),
        in_specs=[
            pl.BlockSpec((BLK, D), lambda j, fi, te: (j, 0)),
            pl.BlockSpec((1, D, FBLK), lambda j, fi, te: (jnp.maximum(te[j], 0), 0, fi)),
            pl.BlockSpec((1, 1, 1, FBLK), lambda j, fi, te: (jnp.maximum(te[j], 0), fi, 0, 0)),
            pl.BlockSpec((1, FBLK, D), lambda j, fi, te: (jnp.maximum(te[j], 0), fi, 0)),
            pl.BlockSpec((1, 1, D), lambda j, fi, te: (jnp.maximum(te[j], 0), 0, 0)),
        ],
        out_specs=pl.BlockSpec((BLK, D), lambda j, fi, te: (j, 0)),
        scratch_shapes=[pltpu.VMEM((BLK, D), jnp.float32)],
    )
    return pl.pallas_call(
        _ffn_body,
        grid_spec=grid_spec,
        out_shape=jax.ShapeDtypeStruct((NPAD, D), jnp.float32),
    )(tile_expert, xs, W1, b1r, W2, b2r)


_SC_MESH = plsc.VectorSubcoreMesh(core_axis_name="c", subcore_axis_name="s")


@functools.partial(
    pl.kernel,
    mesh=_SC_MESH,
    out_type=jax.ShapeDtypeStruct((NPAD, D), jnp.float32),
    scratch_types=[
        pltpu.VMEM((NGB, GB), jnp.int32),
        pltpu.VMEM((GB, D), jnp.float32),
        pltpu.VMEM((GB, D), jnp.float32),
        pltpu.SemaphoreType.DMA,
        pltpu.SemaphoreType.DMA,
    ],
)
def _sc_dispatch(x_hbm, tok_hbm, xs_hbm, idx_v, rows_a, rows_b, sem_a, sem_b):
    wid = lax.axis_index("s") * 2 + lax.axis_index("c")
    base = wid * CHUNK
    pltpu.sync_copy(tok_hbm.at[wid], idx_v)
    bufs = (rows_a, rows_b)
    sems = (sem_a, sem_b)
    cps = [pltpu.async_copy(x_hbm.at[idx_v.at[0]], bufs[0], sems[0]),
           pltpu.async_copy(x_hbm.at[idx_v.at[1]], bufs[1], sems[1])]
    for g in range(NGB):
        cps[g].wait()
        pltpu.sync_copy(bufs[g % 2], xs_hbm.at[pl.ds(base + g * GB, GB)])
        if g + 2 < NGB:
            cps.append(pltpu.async_copy(
                x_hbm.at[idx_v.at[g + 2]], bufs[g % 2], sems[g % 2]))


@functools.partial(
    pl.kernel,
    mesh=_SC_MESH,
    out_type=jax.ShapeDtypeStruct((T, D), jnp.float32),
    scratch_types=[
        pltpu.VMEM((TCH // SCH, SCH), jnp.int32),
        pltpu.VMEM((TCH // SCH, SCH), jnp.int32),
        pltpu.VMEM((TCH, L), jnp.float32),
        pltpu.VMEM((2, SCH, D), jnp.float32),
        pltpu.VMEM((2, SCH, D), jnp.float32),
        pltpu.VMEM((2, SCH, D), jnp.float32),
        pltpu.VMEM((SCH, D), jnp.float32),
        pltpu.SemaphoreType.DMA,
        pltpu.SemaphoreType.DMA,
    ],
)
def _sc_combine(x_hbm, ys_hbm, p0_hbm, p1_hbm, dw_hbm, out_hbm,
                p0_v, p1_v, dw_v, rows0, rows1, xb, ob, sem0, sem1):
    wid = lax.axis_index("s") * 2 + lax.axis_index("c")
    base = wid * TCH
    pltpu.sync_copy(p0_hbm.at[wid], p0_v)
    pltpu.sync_copy(p1_hbm.at[wid], p1_v)
    pltpu.sync_copy(dw_hbm.at[pl.ds(base, TCH)], dw_v)
    nc = TCH // SCH
    sems = (sem0, sem1)

    def fire(c):
        buf = c % 2
        return [pltpu.async_copy(ys_hbm.at[p0_v.at[c]], rows0.at[buf], sems[buf]),
                pltpu.async_copy(ys_hbm.at[p1_v.at[c]], rows1.at[buf], sems[buf]),
                pltpu.async_copy(x_hbm.at[pl.ds(base + c * SCH, SCH)],
                                 xb.at[buf], sems[buf])]

    pend = {0: fire(0), 1: fire(1)}
    for c in range(nc):
        buf = c % 2
        for cp in pend.pop(c):
            cp.wait()

        def body(t, _):
            drow = dw_v[c * SCH + t, :]
            w0r = 1.0 / (1.0 + jnp.exp(-drow))
            w1r = 1.0 / (1.0 + jnp.exp(drow))
            for col in range(D // L):
                cs = pl.ds(col * L, L)
                ob[t, cs] = (xb[buf, t, cs] + w0r * rows0[buf, t, cs]
                             + w1r * rows1[buf, t, cs])
            return 0

        lax.fori_loop(0, SCH, body, 0)
        pltpu.sync_copy(ob, out_hbm.at[pl.ds(base + c * SCH, SCH)])
        if c + 2 < nc:
            pend[c + 2] = fire(c + 2)


def kernel(x, expert_indices, expert_weights, W1, b1, W2, b2):
    x2d = x.reshape(T, D)
    tok_padded, tile_expert, p0, p1 = _prep(expert_indices)

    xs = _sc_dispatch(x2d, tok_padded.reshape(NW, NGB, GB))

    ys = _grouped_ffn(tile_expert, xs, W1, b1, W2, b2)

    dwide = jnp.broadcast_to(
        (expert_weights[..., 0] - expert_weights[..., 1]).reshape(T, 1), (T, L))
    out = _sc_combine(x2d, ys,
                      p0.reshape(NW, TCH // SCH, SCH),
                      p1.reshape(NW, TCH // SCH, SCH),
                      dwide)
    return out.reshape(B, T, D)


# speedup vs baseline: 1.4922x; 1.4222x over previous
import functools

import jax
import jax.numpy as jnp
from jax import lax
from jax.experimental import pallas as pl
from jax.experimental.pallas import tpu as pltpu
from jax.experimental.pallas import tpu_sc as plsc

B, T, D, E, K = 1, 2048, 1024, 8, 2
N = T * K
BLK = 256
NT_MAX = N // BLK + E
NPAD = NT_MAX * BLK
NF = 4
F = 4 * D
FBLK = F // NF

NW = 32
CHUNK = NPAD // NW
GB = 48
NGB = CHUNK // GB
TCH = T // NW
SCH = 16
L = 16


def _prep(expert_indices):
    e_flat = expert_indices.reshape(N).astype(jnp.int32)
    onehot = e_flat[:, None] == jnp.arange(E, dtype=jnp.int32)[None, :]
    oh_i = onehot.astype(jnp.int32)
    counts = jnp.sum(oh_i, axis=0)
    rank = jnp.cumsum(oh_i, axis=0)
    rank_i = jnp.sum(rank * oh_i, axis=1) - 1
    tiles_per_e = (counts + BLK - 1) // BLK
    tile_start_e = jnp.concatenate(
        [jnp.zeros(1, jnp.int32), jnp.cumsum(tiles_per_e, dtype=jnp.int32)])[:E]
    start_i = jnp.sum(jnp.where(onehot, tile_start_e[None, :], 0), axis=1)
    P = start_i * BLK + rank_i
    total_tiles = jnp.sum(tiles_per_e)
    tile_ids = jnp.arange(NT_MAX, dtype=jnp.int32)
    tile_expert = jnp.sum(tile_ids[:, None] >= tile_start_e[None, :],
                          axis=1, dtype=jnp.int32) - 1
    tile_expert = jnp.where(tile_ids < total_tiles, tile_expert, -1)
    p0 = P.reshape(T, K)[:, 0]
    p1 = P.reshape(T, K)[:, 1]
    return tile_expert, p0, p1


def _ffn_body(te_ref, xs_ref, w1_ref, b1_ref, w2_ref, b2_ref,
              out_ref, acc_ref):
    fi = pl.program_id(1)
    j = pl.program_id(0)
    active = te_ref[j] >= 0

    @pl.when(active)
    def _():
        h = jnp.dot(xs_ref[...], w1_ref[0], preferred_element_type=jnp.float32)
        h = h + b1_ref[0, 0, 0][None, :]
        h = h * 0.5 * (1.0 + lax.erf(h * 0.7071067811865476))
        y = jnp.dot(h, w2_ref[0], preferred_element_type=jnp.float32)

        @pl.when(fi == 0)
        def _():
            acc_ref[...] = y

        @pl.when(fi > 0)
        def _():
            acc_ref[...] += y

        @pl.when(fi == NF - 1)
        def _():
            out_ref[...] = acc_ref[...] + b2_ref[0, 0][None, :]


def _grouped_ffn(tile_expert, xs, W1, b1, W2, b2):
    b1r = b1.reshape(E, NF, 1, FBLK)
    b2r = b2.reshape(E, 1, D)
    grid_spec = pltpu.PrefetchScalarGridSpec(
        num_scalar_prefetch=1,
        grid=(NT_MAX, NF),
        in_specs=[
            pl.BlockSpec((BLK, D), lambda j, fi, te: (j, 0)),
            pl.BlockSpec((1, D, FBLK), lambda j, fi, te: (jnp.maximum(te[j], 0), 0, fi)),
            pl.BlockSpec((1, 1, 1, FBLK), lambda j, fi, te: (jnp.maximum(te[j], 0), fi, 0, 0)),
            pl.BlockSpec((1, FBLK, D), lambda j, fi, te: (jnp.maximum(te[j], 0), fi, 0)),
            pl.BlockSpec((1, 1, D), lambda j, fi, te: (jnp.maximum(te[j], 0), 0, 0)),
        ],
        out_specs=pl.BlockSpec((BLK, D), lambda j, fi, te: (j, 0)),
        scratch_shapes=[pltpu.VMEM((BLK, D), jnp.float32)],
    )
    return pl.pallas_call(
        _ffn_body,
        grid_spec=grid_spec,
        out_shape=jax.ShapeDtypeStruct((NPAD, D), jnp.float32),
    )(tile_expert, xs, W1, b1r, W2, b2r)


_SC_MESH = plsc.VectorSubcoreMesh(core_axis_name="c", subcore_axis_name="s")


@functools.partial(
    pl.kernel,
    mesh=_SC_MESH,
    out_type=jax.ShapeDtypeStruct((NPAD, D), jnp.float32),
    scratch_types=[
        pltpu.VMEM((TCH, D), jnp.float32),
        pltpu.VMEM((TCH,), jnp.int32),
        pltpu.VMEM((TCH,), jnp.int32),
        pltpu.SemaphoreType.DMA,
        pltpu.SemaphoreType.DMA,
    ],
)
def _sc_dispatch(x_hbm, p0_hbm, p1_hbm, xs_hbm, xrows, p0_v, p1_v, sem0, sem1):
    wid = lax.axis_index("s") * 2 + lax.axis_index("c")
    tbase = wid * TCH
    pltpu.sync_copy(p0_hbm.at[pl.ds(tbase, TCH)], p0_v)
    pltpu.sync_copy(p1_hbm.at[pl.ds(tbase, TCH)], p1_v)
    pltpu.sync_copy(x_hbm.at[pl.ds(tbase, TCH)], xrows)
    s0 = pltpu.async_copy(xrows, xs_hbm.at[p0_v], sem0)
    s1 = pltpu.async_copy(xrows, xs_hbm.at[p1_v], sem1)
    s0.wait()
    s1.wait()


@functools.partial(
    pl.kernel,
    mesh=_SC_MESH,
    out_type=jax.ShapeDtypeStruct((T, D), jnp.float32),
    scratch_types=[
        pltpu.VMEM((TCH // SCH, SCH), jnp.int32),
        pltpu.VMEM((TCH // SCH, SCH), jnp.int32),
        pltpu.VMEM((TCH, L), jnp.float32),
        pltpu.VMEM((2, SCH, D), jnp.float32),
        pltpu.VMEM((2, SCH, D), jnp.float32),
        pltpu.VMEM((2, SCH, D), jnp.float32),
        pltpu.VMEM((SCH, D), jnp.float32),
        pltpu.SemaphoreType.DMA,
        pltpu.SemaphoreType.DMA,
    ],
)
def _sc_combine(x_hbm, ys_hbm, p0_hbm, p1_hbm, dw_hbm, out_hbm,
                p0_v, p1_v, dw_v, rows0, rows1, xb, ob, sem0, sem1):
    wid = lax.axis_index("s") * 2 + lax.axis_index("c")
    base = wid * TCH
    pltpu.sync_copy(p0_hbm.at[wid], p0_v)
    pltpu.sync_copy(p1_hbm.at[wid], p1_v)
    pltpu.sync_copy(dw_hbm.at[pl.ds(base, TCH)], dw_v)
    nc = TCH // SCH
    sems = (sem0, sem1)

    def fire(c):
        buf = c % 2
        return [pltpu.async_copy(ys_hbm.at[p0_v.at[c]], rows0.at[buf], sems[buf]),
                pltpu.async_copy(ys_hbm.at[p1_v.at[c]], rows1.at[buf], sems[buf]),
                pltpu.async_copy(x_hbm.at[pl.ds(base + c * SCH, SCH)],
                                 xb.at[buf], sems[buf])]

    pend = {0: fire(0), 1: fire(1)}
    for c in range(nc):
        buf = c % 2
        for cp in pend.pop(c):
            cp.wait()

        def body(t, _):
            drow = dw_v[c * SCH + t, :]
            w0r = 1.0 / (1.0 + jnp.exp(-drow))
            w1r = 1.0 / (1.0 + jnp.exp(drow))
            for col in range(D // L):
                cs = pl.ds(col * L, L)
                ob[t, cs] = (xb[buf, t, cs] + w0r * rows0[buf, t, cs]
                             + w1r * rows1[buf, t, cs])
            return 0

        lax.fori_loop(0, SCH, body, 0)
        pltpu.sync_copy(ob, out_hbm.at[pl.ds(base + c * SCH, SCH)])
        if c + 2 < nc:
            pend[c + 2] = fire(c + 2)


def kernel(x, expert_indices, expert_weights, W1, b1, W2, b2):
    x2d = x.reshape(T, D)
    tile_expert, p0, p1 = _prep(expert_indices)

    xs = _sc_dispatch(x2d, p0, p1)

    ys = _grouped_ffn(tile_expert, xs, W1, b1, W2, b2)

    dwide = jnp.broadcast_to(
        (expert_weights[..., 0] - expert_weights[..., 1]).reshape(T, 1), (T, L))
    out = _sc_combine(x2d, ys,
                      p0.reshape(NW, TCH // SCH, SCH),
                      p1.reshape(NW, TCH // SCH, SCH),
                      dwide)
    return out.reshape(B, T, D)


# speedup vs baseline: 1.4982x; 1.0040x over previous
import functools

import jax
import jax.numpy as jnp
from jax import lax
from jax.experimental import pallas as pl
from jax.experimental.pallas import tpu as pltpu
from jax.experimental.pallas import tpu_sc as plsc

B, T, D, E, K = 1, 2048, 1024, 8, 2
N = T * K
BLK = 256
NT_MAX = N // BLK + E
NPAD = NT_MAX * BLK
NF = 4
F = 4 * D
FBLK = F // NF

NW = 32
CHUNK = NPAD // NW
GB = 48
NGB = CHUNK // GB
TCH = T // NW
SCH = 16
L = 16


def _prep(expert_indices):
    e_flat = expert_indices.reshape(N).astype(jnp.int32)
    onehot = e_flat[:, None] == jnp.arange(E, dtype=jnp.int32)[None, :]
    oh_i = onehot.astype(jnp.int32)
    counts = jnp.sum(oh_i, axis=0)
    rank = jnp.cumsum(oh_i, axis=0)
    rank_i = jnp.sum(rank * oh_i, axis=1) - 1
    tiles_per_e = (counts + BLK - 1) // BLK
    tile_start_e = jnp.concatenate(
        [jnp.zeros(1, jnp.int32), jnp.cumsum(tiles_per_e, dtype=jnp.int32)])[:E]
    start_i = jnp.sum(jnp.where(onehot, tile_start_e[None, :], 0), axis=1)
    P = start_i * BLK + rank_i
    total_tiles = jnp.sum(tiles_per_e)
    tile_ids = jnp.arange(NT_MAX, dtype=jnp.int32)
    tile_expert = jnp.sum(tile_ids[:, None] >= tile_start_e[None, :],
                          axis=1, dtype=jnp.int32) - 1
    tile_expert = jnp.where(tile_ids < total_tiles, tile_expert, -1)
    p0 = P.reshape(T, K)[:, 0]
    p1 = P.reshape(T, K)[:, 1]
    return tile_expert, p0, p1


def _ffn_body(te_ref, xs_ref, w1_ref, b1_ref, w2_ref, b2_ref,
              out_ref, acc_ref):
    fi = pl.program_id(1)
    j = pl.program_id(0)
    active = te_ref[j] >= 0

    @pl.when(active)
    def _():
        h = jnp.dot(xs_ref[...], w1_ref[0], preferred_element_type=jnp.float32,
                    precision=lax.Precision.DEFAULT)
        h = h + b1_ref[0, 0, 0][None, :]
        h = h * 0.5 * (1.0 + lax.erf(h * 0.7071067811865476))
        y = jnp.dot(h, w2_ref[0], preferred_element_type=jnp.float32,
                    precision=lax.Precision.DEFAULT)

        @pl.when(fi == 0)
        def _():
            acc_ref[...] = y

        @pl.when(fi > 0)
        def _():
            acc_ref[...] += y

        @pl.when(fi == NF - 1)
        def _():
            out_ref[...] = acc_ref[...] + b2_ref[0, 0][None, :]


def _grouped_ffn(tile_expert, xs, W1, b1, W2, b2):
    b1r = b1.reshape(E, NF, 1, FBLK)
    b2r = b2.reshape(E, 1, D)
    grid_spec = pltpu.PrefetchScalarGridSpec(
        num_scalar_prefetch=1,
        grid=(NT_MAX, NF),
        in_specs=[
            pl.BlockSpec((BLK, D), lambda j, fi, te: (j, 0)),
            pl.BlockSpec((1, D, FBLK), lambda j, fi, te: (jnp.maximum(te[j], 0), 0, fi)),
            pl.BlockSpec((1, 1, 1, FBLK), lambda j, fi, te: (jnp.maximum(te[j], 0), fi, 0, 0)),
            pl.BlockSpec((1, FBLK, D), lambda j, fi, te: (jnp.maximum(te[j], 0), fi, 0)),
            pl.BlockSpec((1, 1, D), lambda j, fi, te: (jnp.maximum(te[j], 0), 0, 0)),
        ],
        out_specs=pl.BlockSpec((BLK, D), lambda j, fi, te: (j, 0)),
        scratch_shapes=[pltpu.VMEM((BLK, D), jnp.float32)],
    )
    return pl.pallas_call(
        _ffn_body,
        grid_spec=grid_spec,
        out_shape=jax.ShapeDtypeStruct((NPAD, D), jnp.float32),
    )(tile_expert, xs, W1, b1r, W2, b2r)


_SC_MESH = plsc.VectorSubcoreMesh(core_axis_name="c", subcore_axis_name="s")


@functools.partial(
    pl.kernel,
    mesh=_SC_MESH,
    out_type=jax.ShapeDtypeStruct((NPAD, D), jnp.float32),
    scratch_types=[
        pltpu.VMEM((TCH, D), jnp.float32),
        pltpu.VMEM((TCH,), jnp.int32),
        pltpu.VMEM((TCH,), jnp.int32),
        pltpu.SemaphoreType.DMA,
        pltpu.SemaphoreType.DMA,
    ],
)
def _sc_dispatch(x_hbm, p0_hbm, p1_hbm, xs_hbm, xrows, p0_v, p1_v, sem0, sem1):
    wid = lax.axis_index("s") * 2 + lax.axis_index("c")
    tbase = wid * TCH
    pltpu.sync_copy(p0_hbm.at[pl.ds(tbase, TCH)], p0_v)
    pltpu.sync_copy(p1_hbm.at[pl.ds(tbase, TCH)], p1_v)
    pltpu.sync_copy(x_hbm.at[pl.ds(tbase, TCH)], xrows)
    s0 = pltpu.async_copy(xrows, xs_hbm.at[p0_v], sem0)
    s1 = pltpu.async_copy(xrows, xs_hbm.at[p1_v], sem1)
    s0.wait()
    s1.wait()


@functools.partial(
    pl.kernel,
    mesh=_SC_MESH,
    out_type=jax.ShapeDtypeStruct((T, D), jnp.float32),
    scratch_types=[
        pltpu.VMEM((TCH // SCH, SCH), jnp.int32),
        pltpu.VMEM((TCH // SCH, SCH), jnp.int32),
        pltpu.VMEM((TCH, L), jnp.float32),
        pltpu.VMEM((2, SCH, D), jnp.float32),
        pltpu.VMEM((2, SCH, D), jnp.float32),
        pltpu.VMEM((2, SCH, D), jnp.float32),
        pltpu.VMEM((SCH, D), jnp.float32),
        pltpu.SemaphoreType.DMA,
        pltpu.SemaphoreType.DMA,
    ],
)
def _sc_combine(x_hbm, ys_hbm, p0_hbm, p1_hbm, dw_hbm, out_hbm,
                p0_v, p1_v, dw_v, rows0, rows1, xb, ob, sem0, sem1):
    wid = lax.axis_index("s") * 2 + lax.axis_index("c")
    base = wid * TCH
    pltpu.sync_copy(p0_hbm.at[wid], p0_v)
    pltpu.sync_copy(p1_hbm.at[wid], p1_v)
    pltpu.sync_copy(dw_hbm.at[pl.ds(base, TCH)], dw_v)
    nc = TCH // SCH
    sems = (sem0, sem1)

    def fire(c):
        buf = c % 2
        return [pltpu.async_copy(ys_hbm.at[p0_v.at[c]], rows0.at[buf], sems[buf]),
                pltpu.async_copy(ys_hbm.at[p1_v.at[c]], rows1.at[buf], sems[buf]),
                pltpu.async_copy(x_hbm.at[pl.ds(base + c * SCH, SCH)],
                                 xb.at[buf], sems[buf])]

    pend = {0: fire(0), 1: fire(1)}
    for c in range(nc):
        buf = c % 2
        for cp in pend.pop(c):
            cp.wait()

        def body(t, _):
            drow = dw_v[c * SCH + t, :]
            w0r = 1.0 / (1.0 + jnp.exp(-drow))
            w1r = 1.0 / (1.0 + jnp.exp(drow))
            for col in range(D // L):
                cs = pl.ds(col * L, L)
                ob[t, cs] = (xb[buf, t, cs] + w0r * rows0[buf, t, cs]
                             + w1r * rows1[buf, t, cs])
            return 0

        lax.fori_loop(0, SCH, body, 0)
        pltpu.sync_copy(ob, out_hbm.at[pl.ds(base + c * SCH, SCH)])
        if c + 2 < nc:
            pend[c + 2] = fire(c + 2)


def kernel(x, expert_indices, expert_weights, W1, b1, W2, b2):
    x2d = x.reshape(T, D)
    tile_expert, p0, p1 = _prep(expert_indices)

    xs = _sc_dispatch(x2d, p0, p1)

    ys = _grouped_ffn(tile_expert, xs, W1, b1, W2, b2)

    dwide = jnp.broadcast_to(
        (expert_weights[..., 0] - expert_weights[..., 1]).reshape(T, 1), (T, L))
    out = _sc_combine(x2d, ys,
                      p0.reshape(NW, TCH // SCH, SCH),
                      p1.reshape(NW, TCH // SCH, SCH),
                      dwide)
    return out.reshape(B, T, D)


# speedup vs baseline: 1.5908x; 1.0619x over previous
import functools

import jax
import jax.numpy as jnp
from jax import lax
from jax.experimental import pallas as pl
from jax.experimental.pallas import tpu as pltpu
from jax.experimental.pallas import tpu_sc as plsc

B, T, D, E, K = 1, 2048, 1024, 8, 2
N = T * K
BLK = 256
NT_MAX = N // BLK + E
NPAD = NT_MAX * BLK
NF = 4
F = 4 * D
FBLK = F // NF

NW = 32
CHUNK = NPAD // NW
GB = 48
NGB = CHUNK // GB
TCH = T // NW
SCH = 16
L = 16


def _prep(expert_indices):
    e_flat = expert_indices.reshape(N).astype(jnp.int32)
    onehot = e_flat[:, None] == jnp.arange(E, dtype=jnp.int32)[None, :]
    oh_i = onehot.astype(jnp.int32)
    counts = jnp.sum(oh_i, axis=0)
    rank = jnp.cumsum(oh_i, axis=0)
    rank_i = jnp.sum(rank * oh_i, axis=1) - 1
    tiles_per_e = (counts + BLK - 1) // BLK
    tile_start_e = jnp.concatenate(
        [jnp.zeros(1, jnp.int32), jnp.cumsum(tiles_per_e, dtype=jnp.int32)])[:E]
    start_i = jnp.sum(jnp.where(onehot, tile_start_e[None, :], 0), axis=1)
    P = start_i * BLK + rank_i
    total_tiles = jnp.sum(tiles_per_e)
    tile_ids = jnp.arange(NT_MAX, dtype=jnp.int32)
    tile_expert = jnp.sum(tile_ids[:, None] >= tile_start_e[None, :],
                          axis=1, dtype=jnp.int32) - 1
    tile_expert = jnp.where(tile_ids < total_tiles, tile_expert, -1)
    p0 = P.reshape(T, K)[:, 0]
    p1 = P.reshape(T, K)[:, 1]
    return tile_expert, p0, p1


def _ffn_body(te_ref, xs_ref, w1_ref, b1_ref, w2_ref, b2_ref,
              out_ref, acc_ref):
    fi = pl.program_id(0)
    j = pl.program_id(1)
    active = te_ref[j] >= 0
    row = pl.ds(j * BLK, BLK)

    @pl.when(active & (fi < NF))
    def _():
        h = jnp.dot(xs_ref[...], w1_ref[0], preferred_element_type=jnp.float32)
        h = h + b1_ref[0, 0, 0][None, :]
        h = h * 0.5 * (1.0 + lax.erf(h * 0.7071067811865476))
        y = jnp.dot(h, w2_ref[0], preferred_element_type=jnp.float32)

        @pl.when(fi == 0)
        def _():
            acc_ref[row, :] = y

        @pl.when(fi > 0)
        def _():
            acc_ref[row, :] += y

    @pl.when(active & (fi == NF))
    def _():
        out_ref[...] = acc_ref[row, :] + b2_ref[0, 0][None, :]


def _grouped_ffn(tile_expert, xs, W1, b1, W2, b2):
    b1r = b1.reshape(E, NF, 1, FBLK)
    b2r = b2.reshape(E, 1, D)

    def te_at(te, j):
        return jnp.maximum(te[j], 0)

    grid_spec = pltpu.PrefetchScalarGridSpec(
        num_scalar_prefetch=1,
        grid=(NF + 1, NT_MAX),
        in_specs=[
            pl.BlockSpec((BLK, D),
                         lambda fi, j, te: (jnp.where(fi < NF, j, 0), 0)),
            pl.BlockSpec((1, D, FBLK),
                         lambda fi, j, te: (jnp.where(fi < NF, te_at(te, j), 0),
                                            0, jnp.where(fi < NF, fi, 0))),
            pl.BlockSpec((1, 1, 1, FBLK),
                         lambda fi, j, te: (jnp.where(fi < NF, te_at(te, j), 0),
                                            jnp.where(fi < NF, fi, 0), 0, 0)),
            pl.BlockSpec((1, FBLK, D),
                         lambda fi, j, te: (jnp.where(fi < NF, te_at(te, j), 0),
                                            jnp.where(fi < NF, fi, 0), 0)),
            pl.BlockSpec((1, 1, D),
                         lambda fi, j, te: (jnp.where(fi < NF, 0, te_at(te, j)),
                                            0, 0)),
        ],
        out_specs=pl.BlockSpec((BLK, D),
                               lambda fi, j, te: (jnp.where(fi < NF, 0, j), 0)),
        scratch_shapes=[pltpu.VMEM((NPAD, D), jnp.float32)],
    )
    return pl.pallas_call(
        _ffn_body,
        grid_spec=grid_spec,
        out_shape=jax.ShapeDtypeStruct((NPAD, D), jnp.float32),
    )(tile_expert, xs, W1, b1r, W2, b2r)


_SC_MESH = plsc.VectorSubcoreMesh(core_axis_name="c", subcore_axis_name="s")


@functools.partial(
    pl.kernel,
    mesh=_SC_MESH,
    out_type=jax.ShapeDtypeStruct((NPAD, D), jnp.float32),
    scratch_types=[
        pltpu.VMEM((TCH, D), jnp.float32),
        pltpu.VMEM((TCH,), jnp.int32),
        pltpu.VMEM((TCH,), jnp.int32),
        pltpu.SemaphoreType.DMA,
        pltpu.SemaphoreType.DMA,
    ],
)
def _sc_dispatch(x_hbm, p0_hbm, p1_hbm, xs_hbm, xrows, p0_v, p1_v, sem0, sem1):
    wid = lax.axis_index("s") * 2 + lax.axis_index("c")
    tbase = wid * TCH
    pltpu.sync_copy(p0_hbm.at[pl.ds(tbase, TCH)], p0_v)
    pltpu.sync_copy(p1_hbm.at[pl.ds(tbase, TCH)], p1_v)
    pltpu.sync_copy(x_hbm.at[pl.ds(tbase, TCH)], xrows)
    s0 = pltpu.async_copy(xrows, xs_hbm.at[p0_v], sem0)
    s1 = pltpu.async_copy(xrows, xs_hbm.at[p1_v], sem1)
    s0.wait()
    s1.wait()


@functools.partial(
    pl.kernel,
    mesh=_SC_MESH,
    out_type=jax.ShapeDtypeStruct((T, D), jnp.float32),
    scratch_types=[
        pltpu.VMEM((TCH // SCH, SCH), jnp.int32),
        pltpu.VMEM((TCH // SCH, SCH), jnp.int32),
        pltpu.VMEM((TCH, L), jnp.float32),
        pltpu.VMEM((2, SCH, D), jnp.float32),
        pltpu.VMEM((2, SCH, D), jnp.float32),
        pltpu.VMEM((2, SCH, D), jnp.float32),
        pltpu.VMEM((SCH, D), jnp.float32),
        pltpu.SemaphoreType.DMA,
        pltpu.SemaphoreType.DMA,
    ],
)
def _sc_combine(x_hbm, ys_hbm, p0_hbm, p1_hbm, dw_hbm, out_hbm,
                p0_v, p1_v, dw_v, rows0, rows1, xb, ob, sem0, sem1):
    wid = lax.axis_index("s") * 2 + lax.axis_index("c")
    base = wid * TCH
    pltpu.sync_copy(p0_hbm.at[wid], p0_v)
    pltpu.sync_copy(p1_hbm.at[wid], p1_v)
    pltpu.sync_copy(dw_hbm.at[pl.ds(base, TCH)], dw_v)
    nc = TCH // SCH
    sems = (sem0, sem1)

    def fire(c):
        buf = c % 2
        return [pltpu.async_copy(ys_hbm.at[p0_v.at[c]], rows0.at[buf], sems[buf]),
                pltpu.async_copy(ys_hbm.at[p1_v.at[c]], rows1.at[buf], sems[buf]),
                pltpu.async_copy(x_hbm.at[pl.ds(base + c * SCH, SCH)],
                                 xb.at[buf], sems[buf])]

    pend = {0: fire(0), 1: fire(1)}
    for c in range(nc):
        buf = c % 2
        for cp in pend.pop(c):
            cp.wait()

        def body(t, _):
            drow = dw_v[c * SCH + t, :]
            w0r = 1.0 / (1.0 + jnp.exp(-drow))
            w1r = 1.0 / (1.0 + jnp.exp(drow))
            for col in range(D // L):
                cs = pl.ds(col * L, L)
                ob[t, cs] = (xb[buf, t, cs] + w0r * rows0[buf, t, cs]
                             + w1r * rows1[buf, t, cs])
            return 0

        lax.fori_loop(0, SCH, body, 0)
        pltpu.sync_copy(ob, out_hbm.at[pl.ds(base + c * SCH, SCH)])
        if c + 2 < nc:
            pend[c + 2] = fire(c + 2)


def kernel(x, expert_indices, expert_weights, W1, b1, W2, b2):
    x2d = x.reshape(T, D)
    tile_expert, p0, p1 = _prep(expert_indices)

    xs = _sc_dispatch(x2d, p0, p1)

    ys = _grouped_ffn(tile_expert, xs, W1, b1, W2, b2)

    dwide = jnp.broadcast_to(
        (expert_weights[..., 0] - expert_weights[..., 1]).reshape(T, 1), (T, L))
    out = _sc_combine(x2d, ys,
                      p0.reshape(NW, TCH // SCH, SCH),
                      p1.reshape(NW, TCH // SCH, SCH),
                      dwide)
    return out.reshape(B, T, D)
